# baseline (speedup 1.0000x reference)
import functools

import jax
import jax.numpy as jnp
from jax import lax
from jax.experimental import pallas as pl
from jax.experimental.pallas import tpu as pltpu
from jax.experimental.pallas import tpu_sc as plsc

N_SC_CORES = 2
N_SUBCORES = 16
N_WORKERS = N_SC_CORES * N_SUBCORES
EDGE_BLK = 128
LANES = 16
FW = 128

_SC_PARAMS = pltpu.CompilerParams(use_tc_tiling_on_sc=False)


def _fill_rows(ref, rows, cols, value):
    v = jnp.full((LANES,), value, jnp.float32)
    per_row = cols // LANES

    def body(i, carry):
        r = i // per_row
        c = (i % per_row) * LANES
        ref[r, pl.ds(c, LANES)] = v
        return carry

    lax.fori_loop(0, rows * per_row, body, 0)


def _fill_flat(ref, n, value):
    v = jnp.full((LANES,), value, jnp.float32)

    def body(i, carry):
        ref[pl.ds(i * LANES, LANES)] = v
        return carry

    lax.fori_loop(0, n // LANES, body, 0)


def _make_degree_pack(N, E):
    n_chunks = E // EDGE_BLK
    rows_pt = n_chunks // N_WORKERS
    extra = n_chunks - rows_pt * N_WORKERS
    NB = 6
    ngroups = rows_pt // NB
    assert rows_pt % NB == 0
    full_ch = 640
    tail = N - (N_SUBCORES - 1) * full_ch
    mesh = plsc.VectorSubcoreMesh(core_axis_name="c", subcore_axis_name="s")

    @functools.partial(
        pl.kernel,
        out_type=(
            jax.ShapeDtypeStruct((N_SC_CORES * N,), jnp.float32),
            jax.ShapeDtypeStruct((n_chunks, EDGE_BLK), jnp.int32),
        ),
        mesh=mesh,
        scratch_types=[
            pltpu.VMEM((rows_pt, EDGE_BLK), jnp.int32),
            pltpu.VMEM((rows_pt, EDGE_BLK), jnp.int32),
            pltpu.VMEM((EDGE_BLK,), jnp.int32),
            pltpu.VMEM((EDGE_BLK,), jnp.int32),
            pltpu.VMEM((EDGE_BLK,), jnp.float32),
            pltpu.VMEM((full_ch,), jnp.float32),
            pltpu.VMEM_SHARED((N,), jnp.float32),
            pltpu.SemaphoreType.DMA,
        ],
        compiler_params=_SC_PARAMS,
    )
    def degpack_kernel(ei_hbm, deg_hbm, packed_hbm,
                       srcs, dsts, ex_s, ex_d, ones_v, zeros_v, acc_sh, sem):
        cid = lax.axis_index("c")
        sid = lax.axis_index("s")
        wid = cid * N_SUBCORES + sid
        _fill_flat(ones_v, EDGE_BLK, 1.0)
        _fill_flat(zeros_v, full_ch, 0.0)
        row0 = wid * rows_pt
        pltpu.sync_copy(ei_hbm.at[0, pl.ds(row0, rows_pt)], srcs)
        pltpu.sync_copy(ei_hbm.at[1, pl.ds(row0, rows_pt)], dsts)

        @pl.when(wid < extra)
        def _():
            pltpu.sync_copy(ei_hbm.at[0, n_chunks - extra + wid], ex_s)
            pltpu.sync_copy(ei_hbm.at[1, n_chunks - extra + wid], ex_d)

        base = sid * full_ch

        @pl.when(sid < N_SUBCORES - 1)
        def _():
            pltpu.sync_copy(zeros_v, acc_sh.at[pl.ds(base, full_ch)])

        @pl.when(sid == N_SUBCORES - 1)
        def _():
            pltpu.sync_copy(zeros_v.at[pl.ds(0, tail)], acc_sh.at[pl.ds(base, tail)])

        def pack_body(i, carry):
            r = i // (EDGE_BLK // LANES)
            c = (i % (EDGE_BLK // LANES)) * LANES
            s = srcs[r, pl.ds(c, LANES)]
            d = dsts[r, pl.ds(c, LANES)]
            srcs[r, pl.ds(c, LANES)] = jnp.left_shift(d, 16) | s
            return carry

        lax.fori_loop(0, rows_pt * (EDGE_BLK // LANES), pack_body, 0)
        pltpu.sync_copy(srcs, packed_hbm.at[pl.ds(row0, rows_pt)])

        @pl.when(wid < extra)
        def _():
            def pack_ex(i, carry):
                s = ex_s[pl.ds(i * LANES, LANES)]
                d = ex_d[pl.ds(i * LANES, LANES)]
                ex_s[pl.ds(i * LANES, LANES)] = jnp.left_shift(d, 16) | s
                return carry

            lax.fori_loop(0, EDGE_BLK // LANES, pack_ex, 0)
            pltpu.sync_copy(ex_s, packed_hbm.at[n_chunks - extra + wid])

        plsc.subcore_barrier()

        def body(g, carry):
            descs = []
            for b in range(NB):
                descs.append(pltpu.async_copy(
                    ones_v, acc_sh.at[dsts.at[g * NB + b]], sem, add=True))
            for d in descs:
                d.wait()
            return carry

        lax.fori_loop(0, ngroups, body, 0)

        @pl.when(wid < extra)
        def _():
            pltpu.sync_copy(ones_v, acc_sh.at[ex_d], add=True)

        plsc.subcore_barrier()

        @pl.when(sid < N_SUBCORES - 1)
        def _():
            pltpu.sync_copy(acc_sh.at[pl.ds(base, full_ch)], zeros_v)
            pltpu.sync_copy(zeros_v, deg_hbm.at[pl.ds(cid * N + base, full_ch)])

        @pl.when(sid == N_SUBCORES - 1)
        def _():
            pltpu.sync_copy(acc_sh.at[pl.ds(base, tail)], zeros_v.at[pl.ds(0, tail)])
            pltpu.sync_copy(zeros_v.at[pl.ds(0, tail)],
                            deg_hbm.at[pl.ds(cid * N + base, tail)])

    return degpack_kernel


def _unpack_idx(packed_ref, row, col0, cb, src_ref, dst_ref):
    def body(i, carry):
        p = packed_ref[row, pl.ds(col0 + i * LANES, LANES)]
        src_ref[pl.ds(i * LANES, LANES)] = p & 0xFFFF
        dst_ref[pl.ds(i * LANES, LANES)] = lax.shift_right_logical(p, 16)
        return carry

    lax.fori_loop(0, cb // LANES, body, 0)


def _make_agg(N, D, E):
    n_rows = E // EDGE_BLK
    rows_pt = n_rows // N_WORKERS
    extra = n_rows - rows_pt * N_WORKERS
    CB = 64
    PC = EDGE_BLK // CB
    chunks_pt = rows_pt * PC
    NB = 4 if D >= 128 else 12
    ngroups = chunks_pt // NB
    assert chunks_pt % NB == 0 and NB >= PC
    rpt = N // N_SUBCORES
    bch = 125
    mesh = plsc.VectorSubcoreMesh(core_axis_name="c", subcore_axis_name="s")

    @functools.partial(
        pl.kernel,
        out_type=jax.ShapeDtypeStruct((N_SC_CORES, N, D), jnp.float32),
        mesh=mesh,
        scratch_types=[
            pltpu.VMEM((rows_pt, EDGE_BLK), jnp.int32),
            pltpu.VMEM((EDGE_BLK,), jnp.int32),
            pltpu.VMEM_SHARED((N, D), jnp.float32),
        ] + [pltpu.VMEM((CB, D), jnp.float32) for _ in range(NB)]
          + [pltpu.VMEM((CB,), jnp.int32) for _ in range(NB)]
          + [pltpu.VMEM((CB,), jnp.int32) for _ in range(NB)]
          + [pltpu.SemaphoreType.DMA for _ in range(NB)]
          + [pltpu.SemaphoreType.DMA],
        compiler_params=_SC_PARAMS,
    )
    def agg_kernel(hs_hbm, packed_hbm, out_hbm, pidx, exp, acc_sh, *rest):
        rows = rest[:NB]
        srcs = rest[NB:2 * NB]
        dsts = rest[2 * NB:3 * NB]
        sem_g = rest[3 * NB:4 * NB]
        sem_s = rest[4 * NB]
        cid = lax.axis_index("c")
        sid = lax.axis_index("s")
        wid = cid * N_SUBCORES + sid

        pltpu.sync_copy(packed_hbm.at[pl.ds(wid * rows_pt, rows_pt)], pidx)
        _fill_rows(rows[0], CB, D, 0.0)
        base = sid * rpt
        for k in range(-(-rpt // CB)):
            n = min(CB, rpt - k * CB)
            pltpu.sync_copy(rows[0].at[pl.ds(0, n)],
                            acc_sh.at[pl.ds(base + k * CB, n)])

        for b in range(NB):
            _unpack_idx(pidx, b // PC, (b % PC) * CB, CB, srcs[b], dsts[b])
            pltpu.async_copy(hs_hbm.at[srcs[b]], rows[b], sem_g[b])
        plsc.subcore_barrier()

        def body(g, carry):
            descs = []
            for b in range(NB):
                pltpu.make_async_copy(hs_hbm.at[srcs[b]], rows[b],
                                      sem_g[b]).wait()
                descs.append(pltpu.async_copy(
                    rows[b], acc_sh.at[dsts[b]], sem_s, add=True))
            for b in range(NB):
                descs[b].wait()

                @pl.when(g + 1 < ngroups)
                def _():
                    q = (g + 1) * NB + b
                    _unpack_idx(pidx, q // PC, (q % PC) * CB, CB,
                                srcs[b], dsts[b])
                    pltpu.async_copy(hs_hbm.at[srcs[b]], rows[b], sem_g[b])
            return carry

        lax.fori_loop(0, ngroups, body, 0)

        @pl.when(wid < extra)
        def _():
            pltpu.sync_copy(packed_hbm.at[n_rows - extra + wid], exp)
            for h in range(PC):
                def unpack_body(i, carry, h=h):
                    p = exp[pl.ds(h * CB + i * LANES, LANES)]
                    srcs[h][pl.ds(i * LANES, LANES)] = p & 0xFFFF
                    dsts[h][pl.ds(i * LANES, LANES)] = (
                        lax.shift_right_logical(p, 16))
                    return carry

                lax.fori_loop(0, CB // LANES, unpack_body, 0)
                pltpu.async_copy(hs_hbm.at[srcs[h]], rows[h], sem_g[h]).wait()
                pltpu.sync_copy(rows[h], acc_sh.at[dsts[h]], add=True)

        plsc.subcore_barrier()
        for k in range(-(-rpt // CB)):
            n = min(CB, rpt - k * CB)
            pltpu.sync_copy(acc_sh.at[pl.ds(base + k * CB, n)],
                            rows[0].at[pl.ds(0, n)])
            pltpu.sync_copy(rows[0].at[pl.ds(0, n)],
                            out_hbm.at[cid, pl.ds(base + k * CB, n)])

    return agg_kernel


def _mm_raw_call(x, W1):
    N, C = x.shape
    H = W1.shape[1]
    BR = 1000

    def body(x_ref, w_ref, o_ref):
        o_ref[...] = jnp.dot(x_ref[...], w_ref[...],
                             preferred_element_type=jnp.float32)

    return pl.pallas_call(
        body,
        grid=(N // BR,),
        in_specs=[
            pl.BlockSpec((BR, C), lambda i: (i, 0)),
            pl.BlockSpec((C, H), lambda i: (0, 0)),
        ],
        out_specs=pl.BlockSpec((BR, H), lambda i: (i, 0)),
        out_shape=jax.ShapeDtypeStruct((N, H), jnp.float32),
    )(x, W1)


def _scale_call(h1, dinv_bc):
    N, H = h1.shape
    BR = 1000

    def body(h_ref, d_ref, hs_ref):
        hs_ref[...] = h_ref[...] * d_ref[...]

    return pl.pallas_call(
        body,
        grid=(N // BR,),
        in_specs=[
            pl.BlockSpec((BR, H), lambda i: (i, 0)),
            pl.BlockSpec((BR, H), lambda i: (i, 0)),
        ],
        out_specs=pl.BlockSpec((BR, H), lambda i: (i, 0)),
        out_shape=jax.ShapeDtypeStruct((N, H), jnp.float32),
    )(h1, dinv_bc)


def _mid_call(agg1, hs1, dinv, b1, W2):
    N, H = hs1.shape
    H2 = W2.shape[1]
    BR = 1000

    def body(a0_ref, a1_ref, hs_ref, d_ref, b_ref, w_ref, o_ref):
        d = d_ref[...]
        z = d * (a0_ref[0] + a1_ref[0] + hs_ref[...]) + b_ref[...]
        r = jnp.maximum(z, 0.0)
        o = jnp.dot(r, w_ref[...], preferred_element_type=jnp.float32)
        o_ref[...] = d[:, :H2] * o

    return pl.pallas_call(
        body,
        grid=(N // BR,),
        in_specs=[
            pl.BlockSpec((1, BR, H), lambda i: (0, i, 0)),
            pl.BlockSpec((1, BR, H), lambda i: (1, i, 0)),
            pl.BlockSpec((BR, H), lambda i: (i, 0)),
            pl.BlockSpec((BR, H), lambda i: (i, 0)),
            pl.BlockSpec((1, H), lambda i: (0, 0)),
            pl.BlockSpec((H, H2), lambda i: (0, 0)),
        ],
        out_specs=pl.BlockSpec((BR, H2), lambda i: (i, 0)),
        out_shape=jax.ShapeDtypeStruct((N, H2), jnp.float32),
    )(agg1, agg1, hs1, dinv, b1, W2)


def _final_call(agg2, hs2, dinv, b2, Wl, bl):
    N, H2 = hs2.shape
    O = Wl.shape[1]
    BR = 1000

    def body(a0_ref, a1_ref, hs_ref, d_ref, b_ref, w_ref, bl_ref, o_ref):
        d = d_ref[:, :H2]
        z = d * (a0_ref[0] + a1_ref[0] + hs_ref[...]) + b_ref[...]
        logits = jnp.dot(z, w_ref[...], preferred_element_type=jnp.float32)
        logits = logits + bl_ref[...]
        m = jnp.max(logits, axis=1, keepdims=True)
        lse = jnp.log(jnp.sum(jnp.exp(logits - m), axis=1, keepdims=True)) + m
        o_ref[...] = logits - lse

    return pl.pallas_call(
        body,
        grid=(N // BR,),
        in_specs=[
            pl.BlockSpec((1, BR, H2), lambda i: (0, i, 0)),
            pl.BlockSpec((1, BR, H2), lambda i: (1, i, 0)),
            pl.BlockSpec((BR, H2), lambda i: (i, 0)),
            pl.BlockSpec((BR, FW), lambda i: (i, 0)),
            pl.BlockSpec((1, H2), lambda i: (0, 0)),
            pl.BlockSpec((H2, O), lambda i: (0, 0)),
            pl.BlockSpec((1, O), lambda i: (0, 0)),
        ],
        out_specs=pl.BlockSpec((BR, O), lambda i: (i, 0)),
        out_shape=jax.ShapeDtypeStruct((N, O), jnp.float32),
    )(agg2, agg2, hs2, dinv, b2, Wl, bl)


def kernel(x, edge_index, W1, b1, W2, b2, Wl, bl):
    N = x.shape[0]
    E = edge_index.shape[1]
    ei3 = edge_index.astype(jnp.int32).reshape(2, E // EDGE_BLK, EDGE_BLK)

    deg, packed = _make_degree_pack(N, E)(ei3)
    h1 = _mm_raw_call(x, W1)
    dinv_vec = lax.rsqrt(deg[:N] + deg[N:] + 1.0)
    dinv = jnp.broadcast_to(dinv_vec[:, None], (N, FW))
    hs1 = _scale_call(h1, dinv)
    agg1 = _make_agg(N, W1.shape[1], E)(hs1, packed)
    hs2 = _mid_call(agg1, hs1, dinv, b1.reshape(1, -1), W2)
    agg2 = _make_agg(N, W2.shape[1], E)(hs2, packed)
    return _final_call(agg2, hs2, dinv, b2.reshape(1, -1), Wl, bl.reshape(1, -1))


# speedup vs baseline: 1.0147x; 1.0147x over previous
import functools

import jax
import jax.numpy as jnp
from jax import lax
from jax.experimental import pallas as pl
from jax.experimental.pallas import tpu as pltpu
from jax.experimental.pallas import tpu_sc as plsc

N_SC_CORES = 2
N_SUBCORES = 16
N_WORKERS = N_SC_CORES * N_SUBCORES
EDGE_BLK = 128
LANES = 16
FW = 128

_SC_PARAMS = pltpu.CompilerParams(use_tc_tiling_on_sc=False)


def _fill_rows(ref, rows, cols, value):
    v = jnp.full((LANES,), value, jnp.float32)
    per_row = cols // LANES

    def body(i, carry):
        r = i // per_row
        c = (i % per_row) * LANES
        ref[r, pl.ds(c, LANES)] = v
        return carry

    lax.fori_loop(0, rows * per_row, body, 0)


def _fill_flat(ref, n, value):
    v = jnp.full((LANES,), value, jnp.float32)

    def body(i, carry):
        ref[pl.ds(i * LANES, LANES)] = v
        return carry

    lax.fori_loop(0, n // LANES, body, 0)


def _make_degree_pack(N, E):
    n_chunks = E // EDGE_BLK
    rows_pt = n_chunks // N_WORKERS
    extra = n_chunks - rows_pt * N_WORKERS
    NB = 6
    ngroups = rows_pt // NB
    assert rows_pt % NB == 0
    full_ch = 640
    tail = N - (N_SUBCORES - 1) * full_ch
    mesh = plsc.VectorSubcoreMesh(core_axis_name="c", subcore_axis_name="s")

    @functools.partial(
        pl.kernel,
        out_type=(
            jax.ShapeDtypeStruct((N_SC_CORES * N,), jnp.float32),
            jax.ShapeDtypeStruct((n_chunks, EDGE_BLK), jnp.int32),
        ),
        mesh=mesh,
        scratch_types=[
            pltpu.VMEM((rows_pt, EDGE_BLK), jnp.int32),
            pltpu.VMEM((rows_pt, EDGE_BLK), jnp.int32),
            pltpu.VMEM((EDGE_BLK,), jnp.int32),
            pltpu.VMEM((EDGE_BLK,), jnp.int32),
            pltpu.VMEM((EDGE_BLK,), jnp.float32),
            pltpu.VMEM((full_ch,), jnp.float32),
            pltpu.VMEM_SHARED((N,), jnp.float32),
            pltpu.SemaphoreType.DMA,
        ],
        compiler_params=_SC_PARAMS,
    )
    def degpack_kernel(ei_hbm, deg_hbm, packed_hbm,
                       srcs, dsts, ex_s, ex_d, ones_v, zeros_v, acc_sh, sem):
        cid = lax.axis_index("c")
        sid = lax.axis_index("s")
        wid = cid * N_SUBCORES + sid
        _fill_flat(ones_v, EDGE_BLK, 1.0)
        _fill_flat(zeros_v, full_ch, 0.0)
        row0 = wid * rows_pt
        pltpu.sync_copy(ei_hbm.at[0, pl.ds(row0, rows_pt)], srcs)
        pltpu.sync_copy(ei_hbm.at[1, pl.ds(row0, rows_pt)], dsts)

        @pl.when(wid < extra)
        def _():
            pltpu.sync_copy(ei_hbm.at[0, n_chunks - extra + wid], ex_s)
            pltpu.sync_copy(ei_hbm.at[1, n_chunks - extra + wid], ex_d)

        base = sid * full_ch

        @pl.when(sid < N_SUBCORES - 1)
        def _():
            pltpu.sync_copy(zeros_v, acc_sh.at[pl.ds(base, full_ch)])

        @pl.when(sid == N_SUBCORES - 1)
        def _():
            pltpu.sync_copy(zeros_v.at[pl.ds(0, tail)], acc_sh.at[pl.ds(base, tail)])

        def pack_body(i, carry):
            r = i // (EDGE_BLK // LANES)
            c = (i % (EDGE_BLK // LANES)) * LANES
            s = srcs[r, pl.ds(c, LANES)]
            d = dsts[r, pl.ds(c, LANES)]
            srcs[r, pl.ds(c, LANES)] = jnp.left_shift(d, 16) | s
            return carry

        lax.fori_loop(0, rows_pt * (EDGE_BLK // LANES), pack_body, 0)
        pltpu.sync_copy(srcs, packed_hbm.at[pl.ds(row0, rows_pt)])

        @pl.when(wid < extra)
        def _():
            def pack_ex(i, carry):
                s = ex_s[pl.ds(i * LANES, LANES)]
                d = ex_d[pl.ds(i * LANES, LANES)]
                ex_s[pl.ds(i * LANES, LANES)] = jnp.left_shift(d, 16) | s
                return carry

            lax.fori_loop(0, EDGE_BLK // LANES, pack_ex, 0)
            pltpu.sync_copy(ex_s, packed_hbm.at[n_chunks - extra + wid])

        plsc.subcore_barrier()

        def body(g, carry):
            descs = []
            for b in range(NB):
                descs.append(pltpu.async_copy(
                    ones_v, acc_sh.at[dsts.at[g * NB + b]], sem, add=True))
            for d in descs:
                d.wait()
            return carry

        lax.fori_loop(0, ngroups, body, 0)

        @pl.when(wid < extra)
        def _():
            pltpu.sync_copy(ones_v, acc_sh.at[ex_d], add=True)

        plsc.subcore_barrier()

        @pl.when(sid < N_SUBCORES - 1)
        def _():
            pltpu.sync_copy(acc_sh.at[pl.ds(base, full_ch)], zeros_v)
            pltpu.sync_copy(zeros_v, deg_hbm.at[pl.ds(cid * N + base, full_ch)])

        @pl.when(sid == N_SUBCORES - 1)
        def _():
            pltpu.sync_copy(acc_sh.at[pl.ds(base, tail)], zeros_v.at[pl.ds(0, tail)])
            pltpu.sync_copy(zeros_v.at[pl.ds(0, tail)],
                            deg_hbm.at[pl.ds(cid * N + base, tail)])

    return degpack_kernel


def _unpack_idx(packed_ref, row, col0, cb, src_ref, dst_ref):
    def body(i, carry):
        p = packed_ref[row, pl.ds(col0 + i * LANES, LANES)]
        src_ref[pl.ds(i * LANES, LANES)] = p & 0xFFFF
        dst_ref[pl.ds(i * LANES, LANES)] = lax.shift_right_logical(p, 16)
        return carry

    lax.fori_loop(0, cb // LANES, body, 0)


def _make_agg(N, D, E):
    n_rows = E // EDGE_BLK
    rows_pt = n_rows // N_WORKERS
    extra = n_rows - rows_pt * N_WORKERS
    CB = 64
    PC = EDGE_BLK // CB
    chunks_pt = rows_pt * PC
    NB = 4 if D >= 128 else 12
    ngroups = chunks_pt // NB
    assert chunks_pt % NB == 0 and NB >= PC
    rpt = N // N_SUBCORES
    bch = 125
    mesh = plsc.VectorSubcoreMesh(core_axis_name="c", subcore_axis_name="s")

    @functools.partial(
        pl.kernel,
        out_type=jax.ShapeDtypeStruct((N_SC_CORES, N, D), jnp.float32),
        mesh=mesh,
        scratch_types=[
            pltpu.VMEM((rows_pt, EDGE_BLK), jnp.int32),
            pltpu.VMEM((EDGE_BLK,), jnp.int32),
            pltpu.VMEM_SHARED((N, D), jnp.float32),
        ] + [pltpu.VMEM((CB, D), jnp.float32) for _ in range(NB)]
          + [pltpu.VMEM((CB,), jnp.int32) for _ in range(NB)]
          + [pltpu.VMEM((CB,), jnp.int32) for _ in range(NB)]
          + [pltpu.SemaphoreType.DMA for _ in range(NB)]
          + [pltpu.SemaphoreType.DMA],
        compiler_params=_SC_PARAMS,
    )
    def agg_kernel(hs_hbm, packed_hbm, out_hbm, pidx, exp, acc_sh, *rest):
        rows = rest[:NB]
        srcs = rest[NB:2 * NB]
        dsts = rest[2 * NB:3 * NB]
        sem_g = rest[3 * NB:4 * NB]
        sem_s = rest[4 * NB]
        cid = lax.axis_index("c")
        sid = lax.axis_index("s")
        wid = cid * N_SUBCORES + sid

        pltpu.sync_copy(packed_hbm.at[pl.ds(wid * rows_pt, rows_pt)], pidx)
        _fill_rows(rows[0], CB, D, 0.0)
        base = sid * rpt
        for k in range(-(-rpt // CB)):
            n = min(CB, rpt - k * CB)
            pltpu.sync_copy(rows[0].at[pl.ds(0, n)],
                            acc_sh.at[pl.ds(base + k * CB, n)])

        for b in range(NB):
            _unpack_idx(pidx, b // PC, (b % PC) * CB, CB, srcs[b], dsts[b])
            pltpu.async_copy(hs_hbm.at[srcs[b]], rows[b], sem_g[b])
        plsc.subcore_barrier()

        def body(g, carry):
            descs = []
            for b in range(NB):
                pltpu.make_async_copy(hs_hbm.at[srcs[b]], rows[b],
                                      sem_g[b]).wait()
                descs.append(pltpu.async_copy(
                    rows[b], acc_sh.at[dsts[b]], sem_s, add=True))
            for b in range(NB):
                descs[b].wait()

                @pl.when(g + 1 < ngroups)
                def _():
                    q = (g + 1) * NB + b
                    _unpack_idx(pidx, q // PC, (q % PC) * CB, CB,
                                srcs[b], dsts[b])
                    pltpu.async_copy(hs_hbm.at[srcs[b]], rows[b], sem_g[b])
            return carry

        lax.fori_loop(0, ngroups, body, 0)

        @pl.when(wid < extra)
        def _():
            pltpu.sync_copy(packed_hbm.at[n_rows - extra + wid], exp)
            for h in range(PC):
                def unpack_body(i, carry, h=h):
                    p = exp[pl.ds(h * CB + i * LANES, LANES)]
                    srcs[h][pl.ds(i * LANES, LANES)] = p & 0xFFFF
                    dsts[h][pl.ds(i * LANES, LANES)] = (
                        lax.shift_right_logical(p, 16))
                    return carry

                lax.fori_loop(0, CB // LANES, unpack_body, 0)
                pltpu.async_copy(hs_hbm.at[srcs[h]], rows[h], sem_g[h]).wait()
                pltpu.sync_copy(rows[h], acc_sh.at[dsts[h]], add=True)

        plsc.subcore_barrier()
        for k in range(-(-rpt // CB)):
            n = min(CB, rpt - k * CB)
            pltpu.sync_copy(acc_sh.at[pl.ds(base + k * CB, n)],
                            rows[0].at[pl.ds(0, n)])
            pltpu.sync_copy(rows[0].at[pl.ds(0, n)],
                            out_hbm.at[cid, pl.ds(base + k * CB, n)])

    return agg_kernel


def _mm_raw_call(x, W1):
    N, C = x.shape
    H = W1.shape[1]
    BR = 1000

    def body(x_ref, w_ref, o_ref):
        o_ref[...] = jnp.dot(x_ref[...], w_ref[...],
                             preferred_element_type=jnp.float32)

    return pl.pallas_call(
        body,
        grid=(N // BR,),
        in_specs=[
            pl.BlockSpec((BR, C), lambda i: (i, 0)),
            pl.BlockSpec((C, H), lambda i: (0, 0)),
        ],
        out_specs=pl.BlockSpec((BR, H), lambda i: (i, 0)),
        out_shape=jax.ShapeDtypeStruct((N, H), jnp.float32),
    )(x, W1)


def _scale_call(h1, deg):
    N, H = h1.shape
    BR = 1000

    def body(h_ref, deg_ref, hs_ref, dinv_ref):
        i = pl.program_id(0)
        d0 = deg_ref[0, i]
        d1 = deg_ref[1, i]
        dvec = lax.rsqrt(d0 + d1 + 1.0)
        dbc = lax.broadcast_in_dim(dvec, (BR, H), (0,))
        dinv_ref[...] = dbc
        hs_ref[...] = h_ref[...] * dbc

    return pl.pallas_call(
        body,
        grid=(N // BR,),
        in_specs=[
            pl.BlockSpec((BR, H), lambda i: (i, 0)),
            pl.BlockSpec((2, N // BR, BR), lambda i: (0, 0, 0)),
        ],
        out_specs=[
            pl.BlockSpec((BR, H), lambda i: (i, 0)),
            pl.BlockSpec((BR, H), lambda i: (i, 0)),
        ],
        out_shape=[
            jax.ShapeDtypeStruct((N, H), jnp.float32),
            jax.ShapeDtypeStruct((N, H), jnp.float32),
        ],
    )(h1, deg)


def _mid_call(agg1, hs1, dinv, b1, W2):
    N, H = hs1.shape
    H2 = W2.shape[1]
    BR = 1000

    def body(a0_ref, a1_ref, hs_ref, d_ref, b_ref, w_ref, o_ref):
        d = d_ref[...]
        z = d * (a0_ref[0] + a1_ref[0] + hs_ref[...]) + b_ref[...]
        r = jnp.maximum(z, 0.0)
        o = jnp.dot(r, w_ref[...], preferred_element_type=jnp.float32)
        o_ref[...] = d[:, :H2] * o

    return pl.pallas_call(
        body,
        grid=(N // BR,),
        in_specs=[
            pl.BlockSpec((1, BR, H), lambda i: (0, i, 0)),
            pl.BlockSpec((1, BR, H), lambda i: (1, i, 0)),
            pl.BlockSpec((BR, H), lambda i: (i, 0)),
            pl.BlockSpec((BR, H), lambda i: (i, 0)),
            pl.BlockSpec((1, H), lambda i: (0, 0)),
            pl.BlockSpec((H, H2), lambda i: (0, 0)),
        ],
        out_specs=pl.BlockSpec((BR, H2), lambda i: (i, 0)),
        out_shape=jax.ShapeDtypeStruct((N, H2), jnp.float32),
    )(agg1, agg1, hs1, dinv, b1, W2)


def _final_call(agg2, hs2, dinv, b2, Wl, bl):
    N, H2 = hs2.shape
    O = Wl.shape[1]
    BR = 1000

    def body(a0_ref, a1_ref, hs_ref, d_ref, b_ref, w_ref, bl_ref, o_ref):
        d = d_ref[:, :H2]
        z = d * (a0_ref[0] + a1_ref[0] + hs_ref[...]) + b_ref[...]
        logits = jnp.dot(z, w_ref[...], preferred_element_type=jnp.float32)
        logits = logits + bl_ref[...]
        m = jnp.max(logits, axis=1, keepdims=True)
        lse = jnp.log(jnp.sum(jnp.exp(logits - m), axis=1, keepdims=True)) + m
        o_ref[...] = logits - lse

    return pl.pallas_call(
        body,
        grid=(N // BR,),
        in_specs=[
            pl.BlockSpec((1, BR, H2), lambda i: (0, i, 0)),
            pl.BlockSpec((1, BR, H2), lambda i: (1, i, 0)),
            pl.BlockSpec((BR, H2), lambda i: (i, 0)),
            pl.BlockSpec((BR, FW), lambda i: (i, 0)),
            pl.BlockSpec((1, H2), lambda i: (0, 0)),
            pl.BlockSpec((H2, O), lambda i: (0, 0)),
            pl.BlockSpec((1, O), lambda i: (0, 0)),
        ],
        out_specs=pl.BlockSpec((BR, O), lambda i: (i, 0)),
        out_shape=jax.ShapeDtypeStruct((N, O), jnp.float32),
    )(agg2, agg2, hs2, dinv, b2, Wl, bl)


def kernel(x, edge_index, W1, b1, W2, b2, Wl, bl):
    N = x.shape[0]
    E = edge_index.shape[1]
    ei3 = edge_index.astype(jnp.int32).reshape(2, E // EDGE_BLK, EDGE_BLK)

    deg, packed = _make_degree_pack(N, E)(ei3)
    h1 = _mm_raw_call(x, W1)
    hs1, dinv = _scale_call(h1, deg.reshape(2, N // 1000, 1000))
    agg1 = _make_agg(N, W1.shape[1], E)(hs1, packed)
    hs2 = _mid_call(agg1, hs1, dinv, b1.reshape(1, -1), W2)
    agg2 = _make_agg(N, W2.shape[1], E)(hs2, packed)
    return _final_call(agg2, hs2, dinv, b2.reshape(1, -1), Wl, bl.reshape(1, -1))
